# baseline (speedup 1.0000x reference)
import functools

import jax
import jax.numpy as jnp
from jax import lax
from jax.experimental import pallas as pl
from jax.experimental.pallas import tpu as pltpu
from jax.experimental.pallas import tpu_sc as plsc

N = 10000
D = 128
E = 320000
NC = 2
NS = 16
NW = NC * NS
CHUNK = 80
SCH = 32
NSB = 4
CPW = NSB * SCH
PADE = NW * CPW * CHUNK
NPAD = 10240
RPT = NPAD // NS
NWBC = RPT // CHUNK
DPT = NPAD // NS

_mesh = plsc.VectorSubcoreMesh(core_axis_name="c", subcore_axis_name="s")


@functools.partial(
    pl.kernel,
    out_type=jax.ShapeDtypeStruct((NC * NPAD,), jnp.float32),
    mesh=_mesh,
    scratch_types=[
        pltpu.VMEM((SCH, CHUNK), jnp.int32),
        pltpu.VMEM((CHUNK,), jnp.float32),
        pltpu.VMEM((DPT,), jnp.float32),
        pltpu.VMEM_SHARED((NPAD,), jnp.float32),
    ],
)
def _deg_kernel(dst_hbm, out_hbm, dstv, ones, wb, acc):
    c = lax.axis_index("c")
    s = lax.axis_index("s")
    w = s * NC + c

    @pl.loop(0, DPT // 16)
    def _zero(i):
        wb[pl.ds(i * 16, 16)] = jnp.zeros((16,), jnp.float32)

    @pl.loop(0, CHUNK // 16)
    def _one(i):
        ones[pl.ds(i * 16, 16)] = jnp.ones((16,), jnp.float32)

    pltpu.sync_copy(wb, acc.at[pl.ds(s * DPT, DPT)])
    plsc.subcore_barrier()

    @pl.loop(0, NSB)
    def _blocks(bk):
        pltpu.sync_copy(dst_hbm.at[w, bk], dstv)

        @pl.loop(0, SCH)
        def _edges(ch):
            pltpu.sync_copy(ones, acc.at[dstv.at[ch]], add=True)

    plsc.subcore_barrier()
    pltpu.sync_copy(acc.at[pl.ds(s * DPT, DPT)], wb)
    pltpu.sync_copy(wb, out_hbm.at[pl.ds(c * NPAD + s * DPT, DPT)])


@functools.partial(
    pl.kernel,
    out_type=jax.ShapeDtypeStruct((NC, NPAD, D), jnp.float32),
    mesh=_mesh,
    scratch_types=[
        pltpu.VMEM((SCH, CHUNK), jnp.int32),
        pltpu.VMEM((SCH, CHUNK), jnp.int32),
        pltpu.VMEM((SCH, CHUNK), jnp.int32),
        pltpu.VMEM((SCH, CHUNK), jnp.int32),
        pltpu.VMEM((CHUNK, D), jnp.float32),
        pltpu.VMEM((CHUNK, D), jnp.float32),
        pltpu.VMEM_SHARED((NPAD, D), jnp.float32),
        pltpu.SemaphoreType.DMA,
        pltpu.SemaphoreType.DMA,
        pltpu.SemaphoreType.DMA,
    ],
)
def _spmm_kernel(xp_hbm, src_hbm, dst_hbm, out_hbm,
                 is0, id0, is1, id1, r0, r1, acc, g0, g1, st):
    c = lax.axis_index("c")
    s = lax.axis_index("s")
    w = s * NC + c

    @pl.loop(0, CHUNK)
    def _zero(r):
        for j in range(D // 16):
            r0[r, pl.ds(j * 16, 16)] = jnp.zeros((16,), jnp.float32)

    for j in range(NWBC):
        pltpu.sync_copy(r0, acc.at[pl.ds(s * RPT + j * CHUNK, CHUNK), :])
    plsc.subcore_barrier()

    @pl.loop(0, NSB)
    def _blocks(bk):
        pltpu.sync_copy(src_hbm.at[w, bk], is0)
        pltpu.sync_copy(dst_hbm.at[w, bk], id0)

        @pl.loop(0, SCH)
        def _edges(ch):
            pltpu.async_copy(xp_hbm.at[is0.at[ch]], r0, g0).wait()
            pltpu.sync_copy(r0, acc.at[id0.at[ch]], add=True)

    plsc.subcore_barrier()
    for j in range(NWBC):
        base = s * RPT + j * CHUNK
        pltpu.sync_copy(acc.at[pl.ds(base, CHUNK), :], r0)
        pltpu.sync_copy(r0, out_hbm.at[c, pl.ds(base, CHUNK), :])


def _scale_body(x_ref, degp_ref, xp_ref, dinv_ref):
    deg = degp_ref[:, 0:1] + degp_ref[:, 1:2] + 1.0
    dinv = lax.rsqrt(deg)
    dinv_ref[...] = dinv
    xp_ref[...] = x_ref[...] * dinv


_scale_call = pl.pallas_call(
    _scale_body,
    out_shape=(
        jax.ShapeDtypeStruct((N, D), jnp.float32),
        jax.ShapeDtypeStruct((N, 1), jnp.float32),
    ),
)


def _dense_body(p_ref, xp_ref, dinv_ref, w1_ref, b1_ref, w2_ref, tp_ref):
    dinv = dinv_ref[...]
    s1 = (p_ref[0, :N] + p_ref[1, :N] + xp_ref[...]) * dinv
    h = jnp.dot(s1, w1_ref[...], preferred_element_type=jnp.float32)
    h = jnp.maximum(h + b1_ref[...].reshape(1, -1), 0.0)
    t = jnp.dot(h, w2_ref[...], preferred_element_type=jnp.float32)
    tp_ref[...] = t * dinv


_dense_call = pl.pallas_call(
    _dense_body,
    out_shape=jax.ShapeDtypeStruct((N, D), jnp.float32),
)


def _softmax_body(q_ref, tp_ref, dinv_ref, b2_ref, o_ref):
    s2 = (q_ref[0, :N] + q_ref[1, :N] + tp_ref[...]) * dinv_ref[...]
    s2 = s2 + b2_ref[...].reshape(1, -1)
    m = jnp.max(s2, axis=1, keepdims=True)
    e = jnp.exp(s2 - m)
    o_ref[...] = e / jnp.sum(e, axis=1, keepdims=True)


_softmax_call = pl.pallas_call(
    _softmax_body,
    out_shape=jax.ShapeDtypeStruct((N, D), jnp.float32),
)


def kernel(x, edge_index, W1, b1, W2, b2):
    ei = edge_index.astype(jnp.int32)
    pad_src = jnp.zeros((PADE - E,), jnp.int32)
    pad_dst = jnp.full((PADE - E,), NPAD - 1, jnp.int32)
    src = jnp.concatenate([ei[0], pad_src]).reshape(NW, NSB, SCH, CHUNK)
    dst = jnp.concatenate([ei[1], pad_dst]).reshape(NW, NSB, SCH, CHUNK)

    deg_p = _deg_kernel(dst).reshape(NC, NPAD)
    deg_p = deg_p[:, :N].T
    xp, dinv = _scale_call(x, deg_p)
    p = _spmm_kernel(xp, src, dst)
    tp = _dense_call(p, xp, dinv, W1, b1, W2)
    q = _spmm_kernel(tp, src, dst)
    return _softmax_call(q, tp, dinv, b2)


# speedup vs baseline: 1.0009x; 1.0009x over previous
import functools

import jax
import jax.numpy as jnp
from jax import lax
from jax.experimental import pallas as pl
from jax.experimental.pallas import tpu as pltpu
from jax.experimental.pallas import tpu_sc as plsc

N = 10000
D = 128
E = 320000
NC = 2
NS = 16
NW = NC * NS
CHUNK = 80
SCH = 32
NSB = 4
CPW = NSB * SCH
PADE = NW * CPW * CHUNK
NPAD = 10240
RPT = NPAD // NS
NWBC = RPT // CHUNK
DPT = NPAD // NS

_mesh = plsc.VectorSubcoreMesh(core_axis_name="c", subcore_axis_name="s")


@functools.partial(
    pl.kernel,
    out_type=jax.ShapeDtypeStruct((NC * NPAD,), jnp.float32),
    mesh=_mesh,
    scratch_types=[
        pltpu.VMEM((SCH, CHUNK), jnp.int32),
        pltpu.VMEM((CHUNK,), jnp.float32),
        pltpu.VMEM((DPT,), jnp.float32),
        pltpu.VMEM_SHARED((NPAD,), jnp.float32),
    ],
)
def _deg_kernel(dst_hbm, out_hbm, dstv, ones, wb, acc):
    c = lax.axis_index("c")
    s = lax.axis_index("s")
    w = s * NC + c

    @pl.loop(0, DPT // 16)
    def _zero(i):
        wb[pl.ds(i * 16, 16)] = jnp.zeros((16,), jnp.float32)

    @pl.loop(0, CHUNK // 16)
    def _one(i):
        ones[pl.ds(i * 16, 16)] = jnp.ones((16,), jnp.float32)

    pltpu.sync_copy(wb, acc.at[pl.ds(s * DPT, DPT)])
    plsc.subcore_barrier()

    @pl.loop(0, NSB)
    def _blocks(bk):
        pltpu.sync_copy(dst_hbm.at[w, bk], dstv)

        @pl.loop(0, SCH)
        def _edges(ch):
            pltpu.sync_copy(ones, acc.at[dstv.at[ch]], add=True)

    plsc.subcore_barrier()
    pltpu.sync_copy(acc.at[pl.ds(s * DPT, DPT)], wb)
    pltpu.sync_copy(wb, out_hbm.at[pl.ds(c * NPAD + s * DPT, DPT)])


@functools.partial(
    pl.kernel,
    out_type=jax.ShapeDtypeStruct((NC, NPAD, D), jnp.float32),
    mesh=_mesh,
    scratch_types=[
        pltpu.VMEM((SCH, CHUNK), jnp.int32),
        pltpu.VMEM((SCH, CHUNK), jnp.int32),
        pltpu.VMEM((SCH, CHUNK), jnp.int32),
        pltpu.VMEM((SCH, CHUNK), jnp.int32),
        pltpu.VMEM((CHUNK, D), jnp.float32),
        pltpu.VMEM((CHUNK, D), jnp.float32),
        pltpu.VMEM_SHARED((NPAD, D), jnp.float32),
        pltpu.SemaphoreType.DMA,
        pltpu.SemaphoreType.DMA,
        pltpu.SemaphoreType.DMA,
    ],
)
def _spmm_kernel(xp_hbm, src_hbm, dst_hbm, out_hbm,
                 is0, id0, is1, id1, r0, r1, acc, g0, g1, st):
    c = lax.axis_index("c")
    s = lax.axis_index("s")
    w = s * NC + c

    @pl.loop(0, CHUNK)
    def _zero(r):
        for j in range(D // 16):
            r0[r, pl.ds(j * 16, 16)] = jnp.zeros((16,), jnp.float32)

    for j in range(NWBC):
        pltpu.sync_copy(r0, acc.at[pl.ds(s * RPT + j * CHUNK, CHUNK), :])
    plsc.subcore_barrier()

    @pl.loop(0, NSB)
    def _blocks(bk):
        pltpu.sync_copy(src_hbm.at[w, bk], is0)
        pltpu.sync_copy(dst_hbm.at[w, bk], id0)

        @pl.loop(0, SCH)
        def _edges(ch):
            pltpu.async_copy(xp_hbm.at[is0.at[ch]], r0, g0).wait()
            pltpu.sync_copy(r0, acc.at[id0.at[ch]], add=True)

    plsc.subcore_barrier()
    for j in range(NWBC):
        base = s * RPT + j * CHUNK
        pltpu.sync_copy(acc.at[pl.ds(base, CHUNK), :], r0)
        pltpu.sync_copy(r0, out_hbm.at[c, pl.ds(base, CHUNK), :])


def _scale_body(x_ref, degp_ref, xp_ref, dinv_ref):
    deg = degp_ref[:, 0:1] + degp_ref[:, 1:2] + 1.0
    dinv = lax.rsqrt(deg)
    dinv_ref[...] = dinv
    xp_ref[...] = x_ref[...] * dinv


_scale_call = pl.pallas_call(
    _scale_body,
    out_shape=(
        jax.ShapeDtypeStruct((N, D), jnp.float32),
        jax.ShapeDtypeStruct((N, 1), jnp.float32),
    ),
)


def _dense_body(p_ref, xp_ref, dinv_ref, w1_ref, b1_ref, w2_ref, tp_ref):
    dinv = dinv_ref[...]
    s1 = (p_ref[0, :N] + p_ref[1, :N] + xp_ref[...]) * dinv
    h = jnp.dot(s1, w1_ref[...], preferred_element_type=jnp.float32)
    h = jnp.maximum(h + b1_ref[...].reshape(1, -1), 0.0)
    t = jnp.dot(h, w2_ref[...], preferred_element_type=jnp.float32)
    tp_ref[...] = t * dinv


_dense_call = pl.pallas_call(
    _dense_body,
    out_shape=jax.ShapeDtypeStruct((N, D), jnp.float32),
)


def _softmax_body(q_ref, tp_ref, dinv_ref, b2_ref, o_ref):
    s2 = (q_ref[0, :N] + q_ref[1, :N] + tp_ref[...]) * dinv_ref[...]
    s2 = s2 + b2_ref[...].reshape(1, -1)
    m = jnp.max(s2, axis=1, keepdims=True)
    e = jnp.exp(s2 - m)
    o_ref[...] = e / jnp.sum(e, axis=1, keepdims=True)


_softmax_call = pl.pallas_call(
    _softmax_body,
    out_shape=jax.ShapeDtypeStruct((N, D), jnp.float32),
)


def kernel(x, edge_index, W1, b1, W2, b2):
    ei = edge_index.astype(jnp.int32)
    pad_src = jnp.zeros((PADE - E,), jnp.int32)
    pad_dst = N + jnp.arange(PADE - E, dtype=jnp.int32) % (NPAD - N)
    src = jnp.concatenate([ei[0], pad_src]).reshape(NW, NSB, SCH, CHUNK)
    dst = jnp.concatenate([ei[1], pad_dst]).reshape(NW, NSB, SCH, CHUNK)

    deg_p = _deg_kernel(dst).reshape(NC, NPAD)
    deg_p = deg_p[:, :N].T
    xp, dinv = _scale_call(x, deg_p)
    p = _spmm_kernel(xp, src, dst)
    tp = _dense_call(p, xp, dinv, W1, b1, W2)
    q = _spmm_kernel(tp, src, dst)
    return _softmax_call(q, tp, dinv, b2)


# speedup vs baseline: 2.2706x; 2.2684x over previous
import functools

import jax
import jax.numpy as jnp
from jax import lax
from jax.experimental import pallas as pl
from jax.experimental.pallas import tpu as pltpu
from jax.experimental.pallas import tpu_sc as plsc

N = 10000
D = 128
E = 320000
NC = 2
NS = 16
NW = NC * NS
CHUNK = 80
SCH = 32
NSB = 4
CPW = NSB * SCH
PADE = NW * CPW * CHUNK
NPAD = 10240
RPT = NPAD // NS
NWBC = RPT // CHUNK
DPT = NPAD // NS

_mesh = plsc.VectorSubcoreMesh(core_axis_name="c", subcore_axis_name="s")


@functools.partial(
    pl.kernel,
    out_type=jax.ShapeDtypeStruct((NC * NPAD,), jnp.float32),
    mesh=_mesh,
    scratch_types=[
        pltpu.VMEM((SCH, CHUNK), jnp.int32),
        pltpu.VMEM((CHUNK,), jnp.float32),
        pltpu.VMEM((DPT,), jnp.float32),
        pltpu.VMEM_SHARED((NPAD,), jnp.float32),
    ],
)
def _deg_kernel(dst_hbm, out_hbm, dstv, ones, wb, acc):
    c = lax.axis_index("c")
    s = lax.axis_index("s")
    w = s * NC + c

    @pl.loop(0, DPT // 16)
    def _zero(i):
        wb[pl.ds(i * 16, 16)] = jnp.zeros((16,), jnp.float32)

    @pl.loop(0, CHUNK // 16)
    def _one(i):
        ones[pl.ds(i * 16, 16)] = jnp.ones((16,), jnp.float32)

    pltpu.sync_copy(wb, acc.at[pl.ds(s * DPT, DPT)])
    plsc.subcore_barrier()

    @pl.loop(0, NSB)
    def _blocks(bk):
        pltpu.sync_copy(dst_hbm.at[w, bk], dstv)

        @pl.loop(0, SCH)
        def _edges(ch):
            pltpu.sync_copy(ones, acc.at[dstv.at[ch]], add=True)

    plsc.subcore_barrier()
    pltpu.sync_copy(acc.at[pl.ds(s * DPT, DPT)], wb)
    pltpu.sync_copy(wb, out_hbm.at[pl.ds(c * NPAD + s * DPT, DPT)])


@functools.partial(
    pl.kernel,
    out_type=jax.ShapeDtypeStruct((NC, NPAD, D), jnp.float32),
    mesh=_mesh,
    scratch_types=[
        pltpu.VMEM((SCH, CHUNK), jnp.int32),
        pltpu.VMEM((SCH, CHUNK), jnp.int32),
        pltpu.VMEM((SCH, CHUNK), jnp.int32),
        pltpu.VMEM((SCH, CHUNK), jnp.int32),
        pltpu.VMEM((CHUNK, D), jnp.float32),
        pltpu.VMEM((CHUNK, D), jnp.float32),
        pltpu.VMEM_SHARED((NPAD, D), jnp.float32),
        pltpu.SemaphoreType.DMA,
        pltpu.SemaphoreType.DMA,
        pltpu.SemaphoreType.DMA,
    ],
)
def _spmm_kernel(xp_hbm, src_hbm, dst_hbm, out_hbm,
                 is0, id0, is1, id1, r0, r1, acc, g0, g1, st):
    c = lax.axis_index("c")
    s = lax.axis_index("s")
    w = s * NC + c

    @pl.loop(0, CHUNK)
    def _zero(r):
        for j in range(D // 16):
            r0[r, pl.ds(j * 16, 16)] = jnp.zeros((16,), jnp.float32)

    for j in range(NWBC):
        pltpu.sync_copy(r0, acc.at[pl.ds(s * RPT + j * CHUNK, CHUNK), :])
    plsc.subcore_barrier()

    @pl.loop(0, NSB)
    def _blocks(bk):
        pltpu.sync_copy(src_hbm.at[w, bk], is0)
        pltpu.sync_copy(dst_hbm.at[w, bk], id0)

        @pl.loop(0, SCH)
        def _edges(ch):
            pltpu.async_copy(xp_hbm.at[is0.at[ch]], r0, g0).wait()
            pltpu.sync_copy(r0, acc.at[id0.at[ch]], add=True)

    plsc.subcore_barrier()
    for j in range(NWBC):
        base = s * RPT + j * CHUNK
        pltpu.sync_copy(acc.at[pl.ds(base, CHUNK), :], r0)
        pltpu.sync_copy(r0, out_hbm.at[c, pl.ds(base, CHUNK), :])


def _scale_body(x_ref, degp_ref, xp_ref, dinv_ref):
    deg = degp_ref[:, 0:1] + degp_ref[:, 1:2] + 1.0
    dinv = lax.rsqrt(deg)
    dinv_ref[...] = dinv
    xp_ref[...] = x_ref[...] * dinv


_scale_call = pl.pallas_call(
    _scale_body,
    out_shape=(
        jax.ShapeDtypeStruct((N, D), jnp.float32),
        jax.ShapeDtypeStruct((N, 1), jnp.float32),
    ),
)


def _dense_body(p_ref, xp_ref, dinv_ref, w1_ref, b1_ref, w2_ref, tp_ref):
    dinv = dinv_ref[...]
    s1 = (p_ref[0, :N] + p_ref[1, :N] + xp_ref[...]) * dinv
    h = jnp.dot(s1, w1_ref[...], preferred_element_type=jnp.float32)
    h = jnp.maximum(h + b1_ref[...].reshape(1, -1), 0.0)
    t = jnp.dot(h, w2_ref[...], preferred_element_type=jnp.float32)
    tp_ref[...] = t * dinv


_dense_call = pl.pallas_call(
    _dense_body,
    out_shape=jax.ShapeDtypeStruct((N, D), jnp.float32),
)


def _softmax_body(q_ref, tp_ref, dinv_ref, b2_ref, o_ref):
    s2 = (q_ref[0, :N] + q_ref[1, :N] + tp_ref[...]) * dinv_ref[...]
    s2 = s2 + b2_ref[...].reshape(1, -1)
    m = jnp.max(s2, axis=1, keepdims=True)
    e = jnp.exp(s2 - m)
    o_ref[...] = e / jnp.sum(e, axis=1, keepdims=True)


_softmax_call = pl.pallas_call(
    _softmax_body,
    out_shape=jax.ShapeDtypeStruct((N, D), jnp.float32),
)


def kernel(x, edge_index, W1, b1, W2, b2):
    ei = edge_index.astype(jnp.int32)
    pad_src = jnp.arange(PADE - E, dtype=jnp.int32) % N
    pad_dst = N + jnp.arange(PADE - E, dtype=jnp.int32) % (NPAD - N)
    src = jnp.concatenate([ei[0], pad_src]).reshape(NW, NSB, SCH, CHUNK)
    dst = jnp.concatenate([ei[1], pad_dst]).reshape(NW, NSB, SCH, CHUNK)

    deg_p = _deg_kernel(dst).reshape(NC, NPAD)
    deg_p = deg_p[:, :N].T
    xp, dinv = _scale_call(x, deg_p)
    p = _spmm_kernel(xp, src, dst)
    tp = _dense_call(p, xp, dinv, W1, b1, W2)
    q = _spmm_kernel(tp, src, dst)
    return _softmax_call(q, tp, dinv, b2)


# speedup vs baseline: 2.9172x; 1.2848x over previous
import functools

import jax
import jax.numpy as jnp
from jax import lax
from jax.experimental import pallas as pl
from jax.experimental.pallas import tpu as pltpu
from jax.experimental.pallas import tpu_sc as plsc

N = 10000
D = 128
E = 320000
NC = 2
NS = 16
NW = NC * NS
CHUNK = 80
SCH = 32
NSB = 4
CPW = NSB * SCH
PADE = NW * CPW * CHUNK
NPAD = 10240
RPT = NPAD // NS
NWBC = RPT // CHUNK
DPT = NPAD // NS

_mesh = plsc.VectorSubcoreMesh(core_axis_name="c", subcore_axis_name="s")


@functools.partial(
    pl.kernel,
    out_type=jax.ShapeDtypeStruct((NC * NPAD,), jnp.float32),
    mesh=_mesh,
    scratch_types=[
        pltpu.VMEM((SCH, CHUNK), jnp.int32),
        pltpu.VMEM((CHUNK,), jnp.float32),
        pltpu.VMEM((DPT,), jnp.float32),
        pltpu.VMEM_SHARED((NPAD,), jnp.float32),
    ],
)
def _deg_kernel(dst_hbm, out_hbm, dstv, ones, wb, acc):
    c = lax.axis_index("c")
    s = lax.axis_index("s")
    w = s * NC + c

    @pl.loop(0, DPT // 16)
    def _zero(i):
        wb[pl.ds(i * 16, 16)] = jnp.zeros((16,), jnp.float32)

    @pl.loop(0, CHUNK // 16)
    def _one(i):
        ones[pl.ds(i * 16, 16)] = jnp.ones((16,), jnp.float32)

    pltpu.sync_copy(wb, acc.at[pl.ds(s * DPT, DPT)])
    plsc.subcore_barrier()

    @pl.loop(0, NSB)
    def _blocks(bk):
        pltpu.sync_copy(dst_hbm.at[w, bk], dstv)

        @pl.loop(0, SCH)
        def _edges(ch):
            pltpu.sync_copy(ones, acc.at[dstv.at[ch]], add=True)

    plsc.subcore_barrier()
    pltpu.sync_copy(acc.at[pl.ds(s * DPT, DPT)], wb)
    pltpu.sync_copy(wb, out_hbm.at[pl.ds(c * NPAD + s * DPT, DPT)])


@functools.partial(
    pl.kernel,
    out_type=jax.ShapeDtypeStruct((NC, NPAD, D), jnp.float32),
    mesh=_mesh,
    scratch_types=[
        pltpu.VMEM((SCH, CHUNK), jnp.int32),
        pltpu.VMEM((SCH, CHUNK), jnp.int32),
        pltpu.VMEM((SCH, CHUNK), jnp.int32),
        pltpu.VMEM((SCH, CHUNK), jnp.int32),
        pltpu.VMEM((CHUNK, D), jnp.float32),
        pltpu.VMEM((CHUNK, D), jnp.float32),
        pltpu.VMEM_SHARED((NPAD, D), jnp.float32),
        pltpu.SemaphoreType.DMA,
        pltpu.SemaphoreType.DMA,
        pltpu.SemaphoreType.DMA,
    ],
)
def _spmm_kernel(xp_hbm, src_hbm, dst_hbm, out_hbm,
                 is0, id0, is1, id1, r0, r1, acc, g0, g1, st):
    c = lax.axis_index("c")
    s = lax.axis_index("s")
    w = s * NC + c

    @pl.loop(0, CHUNK)
    def _zero(r):
        for j in range(D // 16):
            r0[r, pl.ds(j * 16, 16)] = jnp.zeros((16,), jnp.float32)

    for j in range(NWBC):
        pltpu.sync_copy(r0, acc.at[pl.ds(s * RPT + j * CHUNK, CHUNK), :])
    plsc.subcore_barrier()

    def wait_gather(buf, sem):
        pltpu.make_async_copy(xp_hbm.at[pl.ds(0, CHUNK)], buf, sem).wait()

    ibufs = [(is0, id0), (is1, id1)]
    pltpu.sync_copy(src_hbm.at[w, 0], is0)
    pltpu.sync_copy(dst_hbm.at[w, 0], id0)
    stage = None
    for b in range(NSB):
        sb, db = ibufs[b % 2]
        if stage is not None:
            stage[0].wait()
            stage[1].wait()
            stage = None
        if b + 1 < NSB:
            nsb, ndb = ibufs[(b + 1) % 2]
            stage = (
                pltpu.async_copy(src_hbm.at[w, b + 1], nsb, st),
                pltpu.async_copy(dst_hbm.at[w, b + 1], ndb, st),
            )

        pltpu.async_copy(xp_hbm.at[sb.at[0]], r0, g0)

        @pl.loop(0, SCH - 2, step=2)
        def _pipe(i):
            wait_gather(r0, g0)
            pltpu.async_copy(xp_hbm.at[sb.at[i + 1]], r1, g1)
            pltpu.sync_copy(r0, acc.at[db.at[i]], add=True)
            wait_gather(r1, g1)
            pltpu.async_copy(xp_hbm.at[sb.at[i + 2]], r0, g0)
            pltpu.sync_copy(r1, acc.at[db.at[i + 1]], add=True)

        wait_gather(r0, g0)
        pltpu.async_copy(xp_hbm.at[sb.at[SCH - 1]], r1, g1)
        pltpu.sync_copy(r0, acc.at[db.at[SCH - 2]], add=True)
        wait_gather(r1, g1)
        pltpu.sync_copy(r1, acc.at[db.at[SCH - 1]], add=True)

    plsc.subcore_barrier()
    wdesc = [None, None]
    for j in range(NWBC):
        buf, sem = (r0, g0) if j % 2 == 0 else (r1, g1)
        if wdesc[j % 2] is not None:
            wdesc[j % 2].wait()
        base = s * RPT + j * CHUNK
        pltpu.sync_copy(acc.at[pl.ds(base, CHUNK), :], buf)
        wdesc[j % 2] = pltpu.async_copy(
            buf, out_hbm.at[c, pl.ds(base, CHUNK), :], sem)
    wdesc[0].wait()
    wdesc[1].wait()


def _scale_body(x_ref, degp_ref, xp_ref, dinv_ref):
    deg = degp_ref[:, 0:1] + degp_ref[:, 1:2] + 1.0
    dinv = lax.rsqrt(deg)
    dinv_ref[...] = dinv
    xp_ref[...] = x_ref[...] * dinv


_scale_call = pl.pallas_call(
    _scale_body,
    out_shape=(
        jax.ShapeDtypeStruct((N, D), jnp.float32),
        jax.ShapeDtypeStruct((N, 1), jnp.float32),
    ),
)


def _dense_body(p_ref, xp_ref, dinv_ref, w1_ref, b1_ref, w2_ref, tp_ref):
    dinv = dinv_ref[...]
    s1 = (p_ref[0, :N] + p_ref[1, :N] + xp_ref[...]) * dinv
    h = jnp.dot(s1, w1_ref[...], preferred_element_type=jnp.float32)
    h = jnp.maximum(h + b1_ref[...].reshape(1, -1), 0.0)
    t = jnp.dot(h, w2_ref[...], preferred_element_type=jnp.float32)
    tp_ref[...] = t * dinv


_dense_call = pl.pallas_call(
    _dense_body,
    out_shape=jax.ShapeDtypeStruct((N, D), jnp.float32),
)


def _softmax_body(q_ref, tp_ref, dinv_ref, b2_ref, o_ref):
    s2 = (q_ref[0, :N] + q_ref[1, :N] + tp_ref[...]) * dinv_ref[...]
    s2 = s2 + b2_ref[...].reshape(1, -1)
    m = jnp.max(s2, axis=1, keepdims=True)
    e = jnp.exp(s2 - m)
    o_ref[...] = e / jnp.sum(e, axis=1, keepdims=True)


_softmax_call = pl.pallas_call(
    _softmax_body,
    out_shape=jax.ShapeDtypeStruct((N, D), jnp.float32),
)


def kernel(x, edge_index, W1, b1, W2, b2):
    ei = edge_index.astype(jnp.int32)
    pad_src = jnp.arange(PADE - E, dtype=jnp.int32) % N
    pad_dst = N + jnp.arange(PADE - E, dtype=jnp.int32) % (NPAD - N)
    src = jnp.concatenate([ei[0], pad_src]).reshape(NW, NSB, SCH, CHUNK)
    dst = jnp.concatenate([ei[1], pad_dst]).reshape(NW, NSB, SCH, CHUNK)

    deg_p = _deg_kernel(dst).reshape(NC, NPAD)
    deg_p = deg_p[:, :N].T
    xp, dinv = _scale_call(x, deg_p)
    p = _spmm_kernel(xp, src, dst)
    tp = _dense_call(p, xp, dinv, W1, b1, W2)
    q = _spmm_kernel(tp, src, dst)
    return _softmax_call(q, tp, dinv, b2)


# speedup vs baseline: 2.9573x; 1.0137x over previous
import functools

import jax
import jax.numpy as jnp
from jax import lax
from jax.experimental import pallas as pl
from jax.experimental.pallas import tpu as pltpu
from jax.experimental.pallas import tpu_sc as plsc

N = 10000
D = 128
E = 320000
NC = 2
NS = 16
NW = NC * NS
CHUNK = 80
SCH = 32
NSB = 4
CPW = NSB * SCH
PADE = NW * CPW * CHUNK
NPAD = 10240
RPT = NPAD // NS
NWBC = RPT // CHUNK
DPT = NPAD // NS

_mesh = plsc.VectorSubcoreMesh(core_axis_name="c", subcore_axis_name="s")


@functools.partial(
    pl.kernel,
    out_type=jax.ShapeDtypeStruct((NC * NPAD,), jnp.float32),
    mesh=_mesh,
    scratch_types=[
        pltpu.VMEM((SCH, CHUNK), jnp.int32),
        pltpu.VMEM((CHUNK,), jnp.float32),
        pltpu.VMEM((DPT,), jnp.float32),
        pltpu.VMEM_SHARED((NPAD,), jnp.float32),
    ],
)
def _deg_kernel(dst_hbm, out_hbm, dstv, ones, wb, acc):
    c = lax.axis_index("c")
    s = lax.axis_index("s")
    w = s * NC + c

    @pl.loop(0, DPT // 16)
    def _zero(i):
        wb[pl.ds(i * 16, 16)] = jnp.zeros((16,), jnp.float32)

    @pl.loop(0, CHUNK // 16)
    def _one(i):
        ones[pl.ds(i * 16, 16)] = jnp.ones((16,), jnp.float32)

    pltpu.sync_copy(wb, acc.at[pl.ds(s * DPT, DPT)])
    plsc.subcore_barrier()

    @pl.loop(0, NSB)
    def _blocks(bk):
        pltpu.sync_copy(dst_hbm.at[w, bk], dstv)

        @pl.loop(0, SCH)
        def _edges(ch):
            pltpu.sync_copy(ones, acc.at[dstv.at[ch]], add=True)

    plsc.subcore_barrier()
    pltpu.sync_copy(acc.at[pl.ds(s * DPT, DPT)], wb)
    pltpu.sync_copy(wb, out_hbm.at[pl.ds(c * NPAD + s * DPT, DPT)])


@functools.partial(
    pl.kernel,
    out_type=jax.ShapeDtypeStruct((NC, NPAD, D), jnp.float32),
    mesh=_mesh,
    scratch_types=[
        pltpu.VMEM((SCH, CHUNK), jnp.int32),
        pltpu.VMEM((SCH, CHUNK), jnp.int32),
        pltpu.VMEM((SCH, CHUNK), jnp.int32),
        pltpu.VMEM((SCH, CHUNK), jnp.int32),
        pltpu.VMEM((CHUNK, D), jnp.float32),
        pltpu.VMEM((CHUNK, D), jnp.float32),
        pltpu.VMEM_SHARED((NPAD, D), jnp.float32),
        pltpu.SemaphoreType.DMA,
        pltpu.SemaphoreType.DMA,
        pltpu.SemaphoreType.DMA,
        pltpu.SemaphoreType.DMA,
        pltpu.SemaphoreType.DMA,
    ],
)
def _spmm_kernel(xp_hbm, src_hbm, dst_hbm, out_hbm,
                 is0, id0, is1, id1, r0, r1, acc, g0, g1, st, sc0, sc1):
    c = lax.axis_index("c")
    s = lax.axis_index("s")
    w = s * NC + c

    @pl.loop(0, CHUNK)
    def _zero(r):
        for j in range(D // 16):
            r0[r, pl.ds(j * 16, 16)] = jnp.zeros((16,), jnp.float32)

    for j in range(NWBC):
        pltpu.sync_copy(r0, acc.at[pl.ds(s * RPT + j * CHUNK, CHUNK), :])
    plsc.subcore_barrier()

    def wait_gather(buf, sem):
        pltpu.make_async_copy(xp_hbm.at[pl.ds(0, CHUNK)], buf, sem).wait()

    def wait_scatter(sem):
        pltpu.make_async_copy(r0, acc.at[pl.ds(0, CHUNK), :], sem).wait()

    ibufs = [(is0, id0), (is1, id1)]
    pltpu.sync_copy(src_hbm.at[w, 0], is0)
    pltpu.sync_copy(dst_hbm.at[w, 0], id0)
    stage = None
    for b in range(NSB):
        sb, db = ibufs[b % 2]
        if stage is not None:
            stage[0].wait()
            stage[1].wait()
            stage = None
        if b + 1 < NSB:
            nsb, ndb = ibufs[(b + 1) % 2]
            stage = (
                pltpu.async_copy(src_hbm.at[w, b + 1], nsb, st),
                pltpu.async_copy(dst_hbm.at[w, b + 1], ndb, st),
            )

        pltpu.async_copy(xp_hbm.at[sb.at[0]], r0, g0)
        pltpu.async_copy(xp_hbm.at[sb.at[1]], r1, g1)
        wait_gather(r0, g0)
        pltpu.async_copy(r0, acc.at[db.at[0]], sc0, add=True)
        wait_gather(r1, g1)
        pltpu.async_copy(r1, acc.at[db.at[1]], sc1, add=True)

        @pl.loop(2, SCH, step=2)
        def _pipe(i):
            wait_scatter(sc0)
            pltpu.async_copy(xp_hbm.at[sb.at[i]], r0, g0)
            wait_scatter(sc1)
            pltpu.async_copy(xp_hbm.at[sb.at[i + 1]], r1, g1)
            wait_gather(r0, g0)
            pltpu.async_copy(r0, acc.at[db.at[i]], sc0, add=True)
            wait_gather(r1, g1)
            pltpu.async_copy(r1, acc.at[db.at[i + 1]], sc1, add=True)

        wait_scatter(sc0)
        wait_scatter(sc1)

    plsc.subcore_barrier()
    wdesc = [None, None]
    for j in range(NWBC):
        buf, sem = (r0, g0) if j % 2 == 0 else (r1, g1)
        if wdesc[j % 2] is not None:
            wdesc[j % 2].wait()
        base = s * RPT + j * CHUNK
        pltpu.sync_copy(acc.at[pl.ds(base, CHUNK), :], buf)
        wdesc[j % 2] = pltpu.async_copy(
            buf, out_hbm.at[c, pl.ds(base, CHUNK), :], sem)
    wdesc[0].wait()
    wdesc[1].wait()


def _scale_body(x_ref, degp_ref, xp_ref, dinv_ref):
    deg = degp_ref[:, 0:1] + degp_ref[:, 1:2] + 1.0
    dinv = lax.rsqrt(deg)
    dinv_ref[...] = dinv
    xp_ref[...] = x_ref[...] * dinv


_scale_call = pl.pallas_call(
    _scale_body,
    out_shape=(
        jax.ShapeDtypeStruct((N, D), jnp.float32),
        jax.ShapeDtypeStruct((N, 1), jnp.float32),
    ),
)


def _dense_body(p_ref, xp_ref, dinv_ref, w1_ref, b1_ref, w2_ref, tp_ref):
    dinv = dinv_ref[...]
    s1 = (p_ref[0, :N] + p_ref[1, :N] + xp_ref[...]) * dinv
    h = jnp.dot(s1, w1_ref[...], preferred_element_type=jnp.float32)
    h = jnp.maximum(h + b1_ref[...].reshape(1, -1), 0.0)
    t = jnp.dot(h, w2_ref[...], preferred_element_type=jnp.float32)
    tp_ref[...] = t * dinv


_dense_call = pl.pallas_call(
    _dense_body,
    out_shape=jax.ShapeDtypeStruct((N, D), jnp.float32),
)


def _softmax_body(q_ref, tp_ref, dinv_ref, b2_ref, o_ref):
    s2 = (q_ref[0, :N] + q_ref[1, :N] + tp_ref[...]) * dinv_ref[...]
    s2 = s2 + b2_ref[...].reshape(1, -1)
    m = jnp.max(s2, axis=1, keepdims=True)
    e = jnp.exp(s2 - m)
    o_ref[...] = e / jnp.sum(e, axis=1, keepdims=True)


_softmax_call = pl.pallas_call(
    _softmax_body,
    out_shape=jax.ShapeDtypeStruct((N, D), jnp.float32),
)


def kernel(x, edge_index, W1, b1, W2, b2):
    ei = edge_index.astype(jnp.int32)
    pad_src = jnp.arange(PADE - E, dtype=jnp.int32) % N
    pad_dst = N + jnp.arange(PADE - E, dtype=jnp.int32) % (NPAD - N)
    src = jnp.concatenate([ei[0], pad_src]).reshape(NW, NSB, SCH, CHUNK)
    dst = jnp.concatenate([ei[1], pad_dst]).reshape(NW, NSB, SCH, CHUNK)

    deg_p = _deg_kernel(dst).reshape(NC, NPAD)
    deg_p = deg_p[:, :N].T
    xp, dinv = _scale_call(x, deg_p)
    p = _spmm_kernel(xp, src, dst)
    tp = _dense_call(p, xp, dinv, W1, b1, W2)
    q = _spmm_kernel(tp, src, dst)
    return _softmax_call(q, tp, dinv, b2)


# speedup vs baseline: 3.1652x; 1.0703x over previous
import functools

import jax
import jax.numpy as jnp
from jax import lax
from jax.experimental import pallas as pl
from jax.experimental.pallas import tpu as pltpu
from jax.experimental.pallas import tpu_sc as plsc

N = 10000
D = 128
E = 320000
NC = 2
NS = 16
NW = NC * NS
CHUNK = 128
SCH = 20
NSB = 4
CPW = NSB * SCH
PADE = NW * CPW * CHUNK
NPAD = 10240
RPT = NPAD // NS
NWBC = RPT // CHUNK
DPT = NPAD // NS

_mesh = plsc.VectorSubcoreMesh(core_axis_name="c", subcore_axis_name="s")


@functools.partial(
    pl.kernel,
    out_type=jax.ShapeDtypeStruct((NC * NPAD,), jnp.float32),
    mesh=_mesh,
    scratch_types=[
        pltpu.VMEM((SCH, CHUNK), jnp.int32),
        pltpu.VMEM((CHUNK,), jnp.float32),
        pltpu.VMEM((DPT,), jnp.float32),
        pltpu.VMEM_SHARED((NPAD,), jnp.float32),
    ],
)
def _deg_kernel(dst_hbm, out_hbm, dstv, ones, wb, acc):
    c = lax.axis_index("c")
    s = lax.axis_index("s")
    w = s * NC + c

    @pl.loop(0, DPT // 16)
    def _zero(i):
        wb[pl.ds(i * 16, 16)] = jnp.zeros((16,), jnp.float32)

    @pl.loop(0, CHUNK // 16)
    def _one(i):
        ones[pl.ds(i * 16, 16)] = jnp.ones((16,), jnp.float32)

    pltpu.sync_copy(wb, acc.at[pl.ds(s * DPT, DPT)])
    plsc.subcore_barrier()

    @pl.loop(0, NSB)
    def _blocks(bk):
        pltpu.sync_copy(dst_hbm.at[w, bk], dstv)

        @pl.loop(0, SCH)
        def _edges(ch):
            pltpu.sync_copy(ones, acc.at[dstv.at[ch]], add=True)

    plsc.subcore_barrier()
    pltpu.sync_copy(acc.at[pl.ds(s * DPT, DPT)], wb)
    pltpu.sync_copy(wb, out_hbm.at[pl.ds(c * NPAD + s * DPT, DPT)])


@functools.partial(
    pl.kernel,
    out_type=jax.ShapeDtypeStruct((NC, NPAD, D), jnp.float32),
    mesh=_mesh,
    scratch_types=[
        pltpu.VMEM((SCH, CHUNK), jnp.int32),
        pltpu.VMEM((SCH, CHUNK), jnp.int32),
        pltpu.VMEM((SCH, CHUNK), jnp.int32),
        pltpu.VMEM((SCH, CHUNK), jnp.int32),
        pltpu.VMEM((CHUNK, D), jnp.float32),
        pltpu.VMEM((CHUNK, D), jnp.float32),
        pltpu.VMEM_SHARED((NPAD, D), jnp.float32),
        pltpu.SemaphoreType.DMA,
        pltpu.SemaphoreType.DMA,
        pltpu.SemaphoreType.DMA,
        pltpu.SemaphoreType.DMA,
        pltpu.SemaphoreType.DMA,
    ],
)
def _spmm_kernel(xp_hbm, src_hbm, dst_hbm, out_hbm,
                 is0, id0, is1, id1, r0, r1, acc, g0, g1, st, sc0, sc1):
    c = lax.axis_index("c")
    s = lax.axis_index("s")
    w = s * NC + c

    @pl.loop(0, CHUNK)
    def _zero(r):
        for j in range(D // 16):
            r0[r, pl.ds(j * 16, 16)] = jnp.zeros((16,), jnp.float32)

    for j in range(NWBC):
        pltpu.sync_copy(r0, acc.at[pl.ds(s * RPT + j * CHUNK, CHUNK), :])
    plsc.subcore_barrier()

    def wait_gather(buf, sem):
        pltpu.make_async_copy(xp_hbm.at[pl.ds(0, CHUNK)], buf, sem).wait()

    def wait_scatter(sem):
        pltpu.make_async_copy(r0, acc.at[pl.ds(0, CHUNK), :], sem).wait()

    ibufs = [(is0, id0), (is1, id1)]
    pltpu.sync_copy(src_hbm.at[w, 0], is0)
    pltpu.sync_copy(dst_hbm.at[w, 0], id0)
    stage = None
    for b in range(NSB):
        sb, db = ibufs[b % 2]
        if stage is not None:
            stage[0].wait()
            stage[1].wait()
            stage = None
        if b + 1 < NSB:
            nsb, ndb = ibufs[(b + 1) % 2]
            stage = (
                pltpu.async_copy(src_hbm.at[w, b + 1], nsb, st),
                pltpu.async_copy(dst_hbm.at[w, b + 1], ndb, st),
            )

        pltpu.async_copy(xp_hbm.at[sb.at[0]], r0, g0)
        pltpu.async_copy(xp_hbm.at[sb.at[1]], r1, g1)
        wait_gather(r0, g0)
        pltpu.async_copy(r0, acc.at[db.at[0]], sc0, add=True)
        wait_gather(r1, g1)
        pltpu.async_copy(r1, acc.at[db.at[1]], sc1, add=True)

        @pl.loop(2, SCH, step=2)
        def _pipe(i):
            wait_scatter(sc0)
            pltpu.async_copy(xp_hbm.at[sb.at[i]], r0, g0)
            wait_scatter(sc1)
            pltpu.async_copy(xp_hbm.at[sb.at[i + 1]], r1, g1)
            wait_gather(r0, g0)
            pltpu.async_copy(r0, acc.at[db.at[i]], sc0, add=True)
            wait_gather(r1, g1)
            pltpu.async_copy(r1, acc.at[db.at[i + 1]], sc1, add=True)

        wait_scatter(sc0)
        wait_scatter(sc1)

    plsc.subcore_barrier()
    wdesc = [None, None]
    for j in range(NWBC):
        buf, sem = (r0, g0) if j % 2 == 0 else (r1, g1)
        if wdesc[j % 2] is not None:
            wdesc[j % 2].wait()
        base = s * RPT + j * CHUNK
        pltpu.sync_copy(acc.at[pl.ds(base, CHUNK), :], buf)
        wdesc[j % 2] = pltpu.async_copy(
            buf, out_hbm.at[c, pl.ds(base, CHUNK), :], sem)
    wdesc[0].wait()
    wdesc[1].wait()


def _scale_body(x_ref, degp_ref, xp_ref, dinv_ref):
    deg = degp_ref[:, 0:1] + degp_ref[:, 1:2] + 1.0
    dinv = lax.rsqrt(deg)
    dinv_ref[...] = dinv
    xp_ref[...] = x_ref[...] * dinv


_scale_call = pl.pallas_call(
    _scale_body,
    out_shape=(
        jax.ShapeDtypeStruct((N, D), jnp.float32),
        jax.ShapeDtypeStruct((N, 1), jnp.float32),
    ),
)


def _dense_body(p_ref, xp_ref, dinv_ref, w1_ref, b1_ref, w2_ref, tp_ref):
    dinv = dinv_ref[...]
    s1 = (p_ref[0, :N] + p_ref[1, :N] + xp_ref[...]) * dinv
    h = jnp.dot(s1, w1_ref[...], preferred_element_type=jnp.float32)
    h = jnp.maximum(h + b1_ref[...].reshape(1, -1), 0.0)
    t = jnp.dot(h, w2_ref[...], preferred_element_type=jnp.float32)
    tp_ref[...] = t * dinv


_dense_call = pl.pallas_call(
    _dense_body,
    out_shape=jax.ShapeDtypeStruct((N, D), jnp.float32),
)


def _softmax_body(q_ref, tp_ref, dinv_ref, b2_ref, o_ref):
    s2 = (q_ref[0, :N] + q_ref[1, :N] + tp_ref[...]) * dinv_ref[...]
    s2 = s2 + b2_ref[...].reshape(1, -1)
    m = jnp.max(s2, axis=1, keepdims=True)
    e = jnp.exp(s2 - m)
    o_ref[...] = e / jnp.sum(e, axis=1, keepdims=True)


_softmax_call = pl.pallas_call(
    _softmax_body,
    out_shape=jax.ShapeDtypeStruct((N, D), jnp.float32),
)


def kernel(x, edge_index, W1, b1, W2, b2):
    ei = edge_index.astype(jnp.int32)
    pad_src = jnp.arange(PADE - E, dtype=jnp.int32) % N
    pad_dst = N + jnp.arange(PADE - E, dtype=jnp.int32) % (NPAD - N)
    src = jnp.concatenate([ei[0], pad_src]).reshape(NW, NSB, SCH, CHUNK)
    dst = jnp.concatenate([ei[1], pad_dst]).reshape(NW, NSB, SCH, CHUNK)

    deg_p = _deg_kernel(dst).reshape(NC, NPAD)
    deg_p = deg_p[:, :N].T
    xp, dinv = _scale_call(x, deg_p)
    p = _spmm_kernel(xp, src, dst)
    tp = _dense_call(p, xp, dinv, W1, b1, W2)
    q = _spmm_kernel(tp, src, dst)
    return _softmax_call(q, tp, dinv, b2)


# speedup vs baseline: 3.2293x; 1.0203x over previous
import functools

import jax
import jax.numpy as jnp
from jax import lax
from jax.experimental import pallas as pl
from jax.experimental.pallas import tpu as pltpu
from jax.experimental.pallas import tpu_sc as plsc

N = 10000
D = 128
E = 320000
NC = 2
NS = 16
NW = NC * NS
CHUNK = 128
SCH = 20
NSB = 4
CPW = NSB * SCH
PADE = NW * CPW * CHUNK
NPAD = 10240
RPT = NPAD // NS
NWBC = RPT // CHUNK
DPT = NPAD // NS

_mesh = plsc.VectorSubcoreMesh(core_axis_name="c", subcore_axis_name="s")


@functools.partial(
    pl.kernel,
    out_type=jax.ShapeDtypeStruct((NC * NPAD,), jnp.float32),
    mesh=_mesh,
    scratch_types=[
        pltpu.VMEM((SCH, CHUNK), jnp.int32),
        pltpu.VMEM((SCH, CHUNK), jnp.int32),
        pltpu.VMEM((CHUNK,), jnp.float32),
        pltpu.VMEM((DPT,), jnp.float32),
        pltpu.VMEM_SHARED((NPAD,), jnp.float32),
        pltpu.SemaphoreType.DMA,
        pltpu.SemaphoreType.DMA,
    ],
)
def _deg_kernel(dst_hbm, out_hbm, d0, d1, ones, wb, acc, sc, st):
    c = lax.axis_index("c")
    s = lax.axis_index("s")
    w = s * NC + c

    @pl.loop(0, DPT // 16)
    def _zero(i):
        wb[pl.ds(i * 16, 16)] = jnp.zeros((16,), jnp.float32)

    @pl.loop(0, CHUNK // 16)
    def _one(i):
        ones[pl.ds(i * 16, 16)] = jnp.ones((16,), jnp.float32)

    pltpu.sync_copy(wb, acc.at[pl.ds(s * DPT, DPT)])
    plsc.subcore_barrier()

    def drain_scatters():
        @pl.loop(0, SCH)
        def _drain(ch):
            pltpu.make_async_copy(ones, acc.at[pl.ds(0, CHUNK)], sc).wait()

    dbufs = [d0, d1]
    pltpu.sync_copy(dst_hbm.at[w, 0], d0)
    stage = None
    for b in range(NSB):
        db = dbufs[b % 2]
        if stage is not None:
            stage.wait()
            stage = None
        if b + 1 < NSB:
            stage = pltpu.async_copy(dst_hbm.at[w, b + 1], dbufs[(b + 1) % 2], st)

        @pl.loop(0, SCH)
        def _edges(ch):
            pltpu.async_copy(ones, acc.at[db.at[ch]], sc, add=True)

        drain_scatters()

    plsc.subcore_barrier()
    pltpu.sync_copy(acc.at[pl.ds(s * DPT, DPT)], wb)
    pltpu.sync_copy(wb, out_hbm.at[pl.ds(c * NPAD + s * DPT, DPT)])


@functools.partial(
    pl.kernel,
    out_type=jax.ShapeDtypeStruct((NC, NPAD, D), jnp.float32),
    mesh=_mesh,
    scratch_types=[
        pltpu.VMEM((SCH, CHUNK), jnp.int32),
        pltpu.VMEM((SCH, CHUNK), jnp.int32),
        pltpu.VMEM((SCH, CHUNK), jnp.int32),
        pltpu.VMEM((SCH, CHUNK), jnp.int32),
        pltpu.VMEM((CHUNK, D), jnp.float32),
        pltpu.VMEM((CHUNK, D), jnp.float32),
        pltpu.VMEM_SHARED((NPAD, D), jnp.float32),
        pltpu.SemaphoreType.DMA,
        pltpu.SemaphoreType.DMA,
        pltpu.SemaphoreType.DMA,
        pltpu.SemaphoreType.DMA,
        pltpu.SemaphoreType.DMA,
    ],
)
def _spmm_kernel(xp_hbm, src_hbm, dst_hbm, out_hbm,
                 is0, id0, is1, id1, r0, r1, acc, g0, g1, st, sc0, sc1):
    c = lax.axis_index("c")
    s = lax.axis_index("s")
    w = s * NC + c

    @pl.loop(0, CHUNK)
    def _zero(r):
        for j in range(D // 16):
            r0[r, pl.ds(j * 16, 16)] = jnp.zeros((16,), jnp.float32)

    for j in range(NWBC):
        pltpu.sync_copy(r0, acc.at[pl.ds(s * RPT + j * CHUNK, CHUNK), :])
    plsc.subcore_barrier()

    def wait_gather(buf, sem):
        pltpu.make_async_copy(xp_hbm.at[pl.ds(0, CHUNK)], buf, sem).wait()

    def wait_scatter(sem):
        pltpu.make_async_copy(r0, acc.at[pl.ds(0, CHUNK), :], sem).wait()

    ibufs = [(is0, id0), (is1, id1)]
    pltpu.sync_copy(src_hbm.at[w, 0], is0)
    pltpu.sync_copy(dst_hbm.at[w, 0], id0)
    stage = None
    for b in range(NSB):
        sb, db = ibufs[b % 2]
        if stage is not None:
            stage[0].wait()
            stage[1].wait()
            stage = None
        if b + 1 < NSB:
            nsb, ndb = ibufs[(b + 1) % 2]
            stage = (
                pltpu.async_copy(src_hbm.at[w, b + 1], nsb, st),
                pltpu.async_copy(dst_hbm.at[w, b + 1], ndb, st),
            )

        pltpu.async_copy(xp_hbm.at[sb.at[0]], r0, g0)
        pltpu.async_copy(xp_hbm.at[sb.at[1]], r1, g1)
        wait_gather(r0, g0)
        pltpu.async_copy(r0, acc.at[db.at[0]], sc0, add=True)
        wait_gather(r1, g1)
        pltpu.async_copy(r1, acc.at[db.at[1]], sc1, add=True)

        @pl.loop(2, SCH, step=2)
        def _pipe(i):
            wait_scatter(sc0)
            pltpu.async_copy(xp_hbm.at[sb.at[i]], r0, g0)
            wait_scatter(sc1)
            pltpu.async_copy(xp_hbm.at[sb.at[i + 1]], r1, g1)
            wait_gather(r0, g0)
            pltpu.async_copy(r0, acc.at[db.at[i]], sc0, add=True)
            wait_gather(r1, g1)
            pltpu.async_copy(r1, acc.at[db.at[i + 1]], sc1, add=True)

        wait_scatter(sc0)
        wait_scatter(sc1)

    plsc.subcore_barrier()
    wdesc = [None, None]
    for j in range(NWBC):
        buf, sem = (r0, g0) if j % 2 == 0 else (r1, g1)
        if wdesc[j % 2] is not None:
            wdesc[j % 2].wait()
        base = s * RPT + j * CHUNK
        pltpu.sync_copy(acc.at[pl.ds(base, CHUNK), :], buf)
        wdesc[j % 2] = pltpu.async_copy(
            buf, out_hbm.at[c, pl.ds(base, CHUNK), :], sem)
    wdesc[0].wait()
    wdesc[1].wait()


def _scale_body(x_ref, degp_ref, xp_ref, dinv_ref):
    deg = degp_ref[:, 0:1] + degp_ref[:, 1:2] + 1.0
    dinv = lax.rsqrt(deg)
    dinv_ref[...] = dinv
    xp_ref[...] = x_ref[...] * dinv


_scale_call = pl.pallas_call(
    _scale_body,
    out_shape=(
        jax.ShapeDtypeStruct((N, D), jnp.float32),
        jax.ShapeDtypeStruct((N, 1), jnp.float32),
    ),
)


def _dense_body(p_ref, xp_ref, dinv_ref, w1_ref, b1_ref, w2_ref, tp_ref):
    dinv = dinv_ref[...]
    s1 = (p_ref[0, :N] + p_ref[1, :N] + xp_ref[...]) * dinv
    h = jnp.dot(s1, w1_ref[...], preferred_element_type=jnp.float32)
    h = jnp.maximum(h + b1_ref[...].reshape(1, -1), 0.0)
    t = jnp.dot(h, w2_ref[...], preferred_element_type=jnp.float32)
    tp_ref[...] = t * dinv


_dense_call = pl.pallas_call(
    _dense_body,
    out_shape=jax.ShapeDtypeStruct((N, D), jnp.float32),
)


def _softmax_body(q_ref, tp_ref, dinv_ref, b2_ref, o_ref):
    s2 = (q_ref[0, :N] + q_ref[1, :N] + tp_ref[...]) * dinv_ref[...]
    s2 = s2 + b2_ref[...].reshape(1, -1)
    m = jnp.max(s2, axis=1, keepdims=True)
    e = jnp.exp(s2 - m)
    o_ref[...] = e / jnp.sum(e, axis=1, keepdims=True)


_softmax_call = pl.pallas_call(
    _softmax_body,
    out_shape=jax.ShapeDtypeStruct((N, D), jnp.float32),
)


def kernel(x, edge_index, W1, b1, W2, b2):
    ei = edge_index.astype(jnp.int32)
    pad_src = jnp.arange(PADE - E, dtype=jnp.int32) % N
    pad_dst = N + jnp.arange(PADE - E, dtype=jnp.int32) % (NPAD - N)
    src = jnp.concatenate([ei[0], pad_src]).reshape(NW, NSB, SCH, CHUNK)
    dst = jnp.concatenate([ei[1], pad_dst]).reshape(NW, NSB, SCH, CHUNK)

    deg_p = _deg_kernel(dst).reshape(NC, NPAD)
    deg_p = deg_p[:, :N].T
    xp, dinv = _scale_call(x, deg_p)
    p = _spmm_kernel(xp, src, dst)
    tp = _dense_call(p, xp, dinv, W1, b1, W2)
    q = _spmm_kernel(tp, src, dst)
    return _softmax_call(q, tp, dinv, b2)
